# baseline (speedup 1.0000x reference)
import jax
import jax.numpy as jnp
from jax import lax
from jax.experimental import pallas as pl
from jax.experimental.pallas import tpu as pltpu
from jax.experimental.pallas import tpu_sc as plsc

_VOCAB = 100000
_M = 10
_K = 256
_SUB = 30
_NTOK = 4096 * 50
_NW = 32
_TPW = _NTOK // _NW
_CH = 64
_NCH = _TPW // _CH
_CBW = _M * _K * _SUB
_D = _M * _SUB


def _sc_body(ids_hbm, vec_hbm, cw_hbm, out_hbm, cb_v, ids_v, codes_v, out_v, gsem):
    cid = lax.axis_index("c")
    sid = lax.axis_index("s")
    wid = sid * 2 + cid
    tok0 = wid * _TPW

    pltpu.sync_copy(cw_hbm, cb_v)
    pltpu.sync_copy(ids_hbm.at[pl.ds(tok0, _TPW)], ids_v)
    lane = lax.iota(jnp.int32, 16)

    def chunk(c, carry):
        base = c * _CH
        pltpu.async_copy(vec_hbm.at[ids_v.at[pl.ds(base, _CH)]], codes_v, gsem).wait()

        def grp(g2, carry2):
            rows = g2 * 16 + lane

            def sub(i, carry3):
                col = jnp.full((16,), i, jnp.int32)
                c16 = plsc.load_gather(codes_v, [rows, col])
                bvec = c16 * _SUB + i * (_K * _SUB)
                col0 = i * _SUB
                for d in range(_SUB):
                    vals = plsc.load_gather(cb_v, [bvec + d])
                    plsc.store_scatter(
                        out_v, [rows, jnp.full((16,), col0 + d, jnp.int32)], vals)
                return carry3

            return lax.fori_loop(0, _M, sub, carry2)

        lax.fori_loop(0, _CH // 16, grp, carry)
        pltpu.sync_copy(out_v, out_hbm.at[pl.ds(tok0 + base, _CH)])
        return carry

    lax.fori_loop(0, _NCH, chunk, 0)


def kernel(input_ids, codewords, vectors):
    ids = input_ids.reshape(_NTOK)
    cw = codewords.reshape(_CBW)
    mesh = plsc.VectorSubcoreMesh(core_axis_name="c", subcore_axis_name="s")
    out = pl.kernel(
        _sc_body,
        out_type=jax.ShapeDtypeStruct((_NTOK, _D), jnp.float32),
        mesh=mesh,
        compiler_params=pltpu.CompilerParams(
            use_tc_tiling_on_sc=False, needs_layout_passes=False),
        scratch_types=[
            pltpu.VMEM((_CBW,), jnp.float32),
            pltpu.VMEM((_TPW,), jnp.int32),
            pltpu.VMEM((_CH, _M), jnp.int32),
            pltpu.VMEM((_CH, _D), jnp.float32),
            pltpu.SemaphoreType.DMA,
        ],
    )(ids, vectors, cw)
    return out.reshape(4096, 50, _D)


# speedup vs baseline: 1.1211x; 1.1211x over previous
import jax
import jax.numpy as jnp
from jax import lax
from jax.experimental import pallas as pl
from jax.experimental.pallas import tpu as pltpu
from jax.experimental.pallas import tpu_sc as plsc

_VOCAB = 100000
_M = 10
_K = 256
_SUB = 30
_NTOK = 4096 * 50
_NW = 32
_TPW = _NTOK // _NW
_CH = 64
_NCH = _TPW // _CH
_CBW = _M * _K * _SUB
_D = _M * _SUB


def _sc_body(ids_hbm, vec_hbm, cw_hbm, out_hbm,
             cb_v, ids_v, codes0, codes1, out0, out1,
             sg0, sg1, so0, so1):
    cid = lax.axis_index("c")
    sid = lax.axis_index("s")
    wid = sid * 2 + cid
    tok0 = wid * _TPW
    codes_b = (codes0, codes1)
    out_b = (out0, out1)
    sg = (sg0, sg1)
    so = (so0, so1)

    pltpu.sync_copy(cw_hbm, cb_v)
    pltpu.sync_copy(ids_hbm.at[pl.ds(tok0, _TPW)], ids_v)
    lane = lax.iota(jnp.int32, 16)

    def gather_codes(c, p):
        pltpu.async_copy(
            vec_hbm.at[ids_v.at[pl.ds(c * _CH, _CH)]], codes_b[p], sg[p])

    gather_codes(0, 0)

    def half(cp, p):
        c = cp * 2 + p
        pltpu.make_async_copy(
            vec_hbm.at[ids_v.at[pl.ds(c * _CH, _CH)]], codes_b[p], sg[p]).wait()
        gather_codes(jnp.minimum(c + 1, _NCH - 1), 1 - p)
        @pl.when(cp >= 1)
        def _():
            pltpu.make_async_copy(
                out_b[p], out_hbm.at[pl.ds(tok0, _CH)], so[p]).wait()

        def grp(g2, carry):
            rows = g2 * 16 + lane
            for i in range(_M):
                c16 = plsc.load_gather(
                    codes_b[p], [rows, jnp.full((16,), i, jnp.int32)])
                bvec = c16 * _SUB + (i * _K * _SUB)
                base = i * _SUB
                for d in range(_SUB):
                    vals = plsc.load_gather(cb_v, [bvec + d])
                    plsc.store_scatter(
                        out_b[p], [rows, jnp.full((16,), base + d, jnp.int32)],
                        vals)
            return carry

        lax.fori_loop(0, _CH // 16, grp, 0)
        pltpu.async_copy(out_b[p], out_hbm.at[pl.ds(tok0 + c * _CH, _CH)], so[p])

    def pair(cp, carry):
        half(cp, 0)
        half(cp, 1)
        return carry

    lax.fori_loop(0, _NCH // 2, pair, 0)
    pltpu.make_async_copy(
        vec_hbm.at[ids_v.at[pl.ds(0, _CH)]], codes_b[0], sg[0]).wait()
    for p in (0, 1):
        pltpu.make_async_copy(
            out_b[p], out_hbm.at[pl.ds(tok0, _CH)], so[p]).wait()


def kernel(input_ids, codewords, vectors):
    ids = input_ids.reshape(_NTOK)
    cw = codewords.reshape(_CBW)
    vec16 = jnp.pad(vectors, ((0, 0), (0, 16 - _M)))
    mesh = plsc.VectorSubcoreMesh(core_axis_name="c", subcore_axis_name="s")
    out = pl.kernel(
        _sc_body,
        out_type=jax.ShapeDtypeStruct((_NTOK, _D), jnp.float32),
        mesh=mesh,
        compiler_params=pltpu.CompilerParams(
            use_tc_tiling_on_sc=False, needs_layout_passes=False),
        scratch_types=[
            pltpu.VMEM((_CBW,), jnp.float32),
            pltpu.VMEM((_TPW,), jnp.int32),
            pltpu.VMEM((_CH, 16), jnp.int32),
            pltpu.VMEM((_CH, 16), jnp.int32),
            pltpu.VMEM((_CH, _D), jnp.float32),
            pltpu.VMEM((_CH, _D), jnp.float32),
            pltpu.SemaphoreType.DMA,
            pltpu.SemaphoreType.DMA,
            pltpu.SemaphoreType.DMA,
            pltpu.SemaphoreType.DMA,
        ],
    )(ids, vec16, cw)
    return out.reshape(4096, 50, _D)


# speedup vs baseline: 1.4827x; 1.3226x over previous
import jax
import jax.numpy as jnp
from jax import lax
from jax.experimental import pallas as pl
from jax.experimental.pallas import tpu as pltpu
from jax.experimental.pallas import tpu_sc as plsc

_VOCAB = 100000
_M = 10
_K = 256
_SUB = 30
_NTOK = 4096 * 50
_NW = 32
_TPW = _NTOK // _NW
_CH = 64
_NCH = _TPW // _CH
_CBW = _M * _K * _SUB
_D = _M * _SUB


def _sc_body(ids_hbm, vec_hbm, cw_hbm, out_hbm,
             cb_v, ids_v, codes0, codes1, out0, out1,
             sg0, sg1, so0, so1):
    cid = lax.axis_index("c")
    sid = lax.axis_index("s")
    wid = sid * 2 + cid
    tok0 = wid * _TPW
    codes_b = (codes0, codes1)
    out_b = (out0, out1)
    sg = (sg0, sg1)
    so = (so0, so1)

    pltpu.sync_copy(cw_hbm, cb_v)
    pltpu.sync_copy(ids_hbm.at[pl.ds(tok0, _TPW)], ids_v)
    lane = lax.iota(jnp.int32, 16)

    def gather_codes(c, p):
        pltpu.async_copy(
            vec_hbm.at[ids_v.at[pl.ds(c * _CH, _CH)]], codes_b[p], sg[p])

    gather_codes(0, 0)

    def half(cp, p):
        c = cp * 2 + p
        pltpu.make_async_copy(
            vec_hbm.at[ids_v.at[pl.ds(c * _CH, _CH)]], codes_b[p], sg[p]).wait()
        gather_codes(jnp.minimum(c + 1, _NCH - 1), 1 - p)
        @pl.when(cp >= 1)
        def _():
            pltpu.make_async_copy(
                out_b[p], out_hbm.at[pl.ds(tok0, _CH)], so[p]).wait()

        def grp(g2, carry):
            rows = g2 * 16 + lane
            for i in range(_M):
                c16 = plsc.load_gather(
                    codes_b[p], [rows, jnp.full((16,), i, jnp.int32)])
                bvec = c16 * _SUB + (i * _K * _SUB)
                col0 = jnp.full((16,), i * _SUB, jnp.int32)

                @plsc.parallel_loop(0, _SUB, unroll=6)
                def _dloop(d):
                    vals = plsc.load_gather(cb_v, [bvec + d])
                    plsc.store_scatter(out_b[p], [rows, col0 + d], vals)
            return carry

        lax.fori_loop(0, _CH // 16, grp, 0)
        pltpu.async_copy(out_b[p], out_hbm.at[pl.ds(tok0 + c * _CH, _CH)], so[p])

    def pair(cp, carry):
        half(cp, 0)
        half(cp, 1)
        return carry

    lax.fori_loop(0, _NCH // 2, pair, 0)
    pltpu.make_async_copy(
        vec_hbm.at[ids_v.at[pl.ds(0, _CH)]], codes_b[0], sg[0]).wait()
    for p in (0, 1):
        pltpu.make_async_copy(
            out_b[p], out_hbm.at[pl.ds(tok0, _CH)], so[p]).wait()


def kernel(input_ids, codewords, vectors):
    ids = input_ids.reshape(_NTOK)
    cw = codewords.reshape(_CBW)
    vec16 = jnp.pad(vectors, ((0, 0), (0, 16 - _M)))
    mesh = plsc.VectorSubcoreMesh(core_axis_name="c", subcore_axis_name="s")
    out = pl.kernel(
        _sc_body,
        out_type=jax.ShapeDtypeStruct((_NTOK, _D), jnp.float32),
        mesh=mesh,
        compiler_params=pltpu.CompilerParams(
            use_tc_tiling_on_sc=False, needs_layout_passes=False),
        scratch_types=[
            pltpu.VMEM((_CBW,), jnp.float32),
            pltpu.VMEM((_TPW,), jnp.int32),
            pltpu.VMEM((_CH, 16), jnp.int32),
            pltpu.VMEM((_CH, 16), jnp.int32),
            pltpu.VMEM((_CH, _D), jnp.float32),
            pltpu.VMEM((_CH, _D), jnp.float32),
            pltpu.SemaphoreType.DMA,
            pltpu.SemaphoreType.DMA,
            pltpu.SemaphoreType.DMA,
            pltpu.SemaphoreType.DMA,
        ],
    )(ids, vec16, cw)
    return out.reshape(4096, 50, _D)
